# baseline (speedup 1.0000x reference)
import functools

import jax
import jax.numpy as jnp
from jax import lax
from jax.experimental import pallas as pl
from jax.experimental.pallas import tpu as pltpu
from jax.experimental.pallas import tpu_sc as plsc

NC = 2
NS = 16
NW = NC * NS

CH = 128
SBLK = 512
CPB = SBLK // CH
WBC = 80


def _rel_transform_body(x_ref, w_ref, o_ref):
  o_ref[...] = lax.dot_general(
      x_ref[...], w_ref[0],
      dimension_numbers=(((1,), (1,)), ((), ())),
      preferred_element_type=jnp.float32,
  )


def _self_transform_body(x_ref, w_ref, b_ref, o_ref):
  o_ref[...] = lax.dot_general(
      x_ref[...], w_ref[...],
      dimension_numbers=(((1,), (1,)), ((), ())),
      preferred_element_type=jnp.float32,
  ) + b_ref[...]


def _combine_body(p_ref, s_ref, o_ref):
  o_ref[...] = p_ref[0] + p_ref[1] + s_ref[...]


def _make_sc_kernel(n, e, d, r):
  nbt = e // SBLK
  nba = -(-nbt // NS)
  nbw = -(-nbt // NW)
  remw = nbt % NW
  nwmax = nbw * CPB
  deg_sz = r * n
  nrc = n // WBC
  ndz = deg_sz // WBC

  mesh = plsc.VectorSubcoreMesh(core_axis_name="c", subcore_axis_name="s")

  @functools.partial(
      pl.kernel,
      mesh=mesh,
      compiler_params=pltpu.CompilerParams(needs_layout_passes=False),
      out_type=jax.ShapeDtypeStruct((NC, n, d), jnp.float32),
      scratch_types=[
          pltpu.VMEM_SHARED((deg_sz,), jnp.float32),
          pltpu.VMEM_SHARED((n, d), jnp.float32),
          pltpu.VMEM((2, 4, SBLK), jnp.float32),
          pltpu.VMEM((2, 2, SBLK), jnp.int32),
          pltpu.VMEM((2, SBLK), jnp.int32),
          pltpu.VMEM((2 * CPB, CH), jnp.int32),
          pltpu.VMEM((2 * CPB, CH), jnp.float32),
          pltpu.VMEM((2, CH, d), jnp.float32),
          pltpu.VMEM((2, CH), jnp.float32),
          pltpu.VMEM((2, CH), jnp.int32),
          pltpu.VMEM((CH,), jnp.float32),
          pltpu.SemaphoreType.DMA((2,)),
          pltpu.SemaphoreType.DMA((2,)),
          pltpu.SemaphoreType.DMA((2,)),
          pltpu.SemaphoreType.DMA,
      ],
  )
  def sc_kernel(trel, et_h, ei_h, ef_t, out_hbm,
                deg_sp, out_sp,
                ef4s, eits, ets2, key_a, wabs_a,
                rows2, c2v, srow2, degc,
                sem_g, sem_s, sem_a, sem_b):
    cid = lax.axis_index("c")
    sid = lax.axis_index("s")
    wid = sid * NC + cid
    z16 = jnp.zeros((16,), jnp.float32)

    def fire_stage(blk, pb):
      off = blk * SBLK
      pltpu.async_copy(ei_h.at[:, pl.ds(off, SBLK)], eits.at[pb], sem_b)
      pltpu.async_copy(ef_t.at[:, pl.ds(off, SBLK)], ef4s.at[pb], sem_b)
      pltpu.async_copy(et_h.at[pl.ds(off, SBLK)], ets2.at[pb], sem_b)

    def wait_stage(pb):
      pltpu.make_async_copy(ei_h.at[:, pl.ds(0, SBLK)], eits.at[pb],
                            sem_b).wait()
      pltpu.make_async_copy(ef_t.at[:, pl.ds(0, SBLK)], ef4s.at[pb],
                            sem_b).wait()
      pltpu.make_async_copy(et_h.at[pl.ds(0, SBLK)], ets2.at[pb],
                            sem_b).wait()

    def drain_ascatters(par):
      for jj in range(CPB):
        pltpu.make_async_copy(wabs_a.at[par * CPB + jj],
                              deg_sp.at[key_a.at[par * CPB + jj]],
                              sem_a.at[par]).wait()

    def zrow(i, carry):
      for h in range(d // 16):
        rows2[0, i, pl.ds(h * 16, 16)] = z16
      return carry

    lax.fori_loop(0, CH, zrow, 0)

    def zout(k, carry):
      ch_id = sid + k * NS

      @pl.when(ch_id < nrc)
      def _():
        pltpu.sync_copy(rows2.at[0, pl.ds(0, WBC)],
                        out_sp.at[pl.ds(ch_id * WBC, WBC)])

      return carry

    lax.fori_loop(0, -(-nrc // NS), zout, 0)

    def zdeg(k, carry):
      ch_id = sid + k * NS

      @pl.when(ch_id < ndz)
      def _():
        pltpu.sync_copy(rows2.at[0, 0, pl.ds(0, WBC)],
                        deg_sp.at[pl.ds(ch_id * WBC, WBC)])

      return carry

    lax.fori_loop(0, -(-ndz // NS), zdeg, 0)
    fire_stage(sid, 0)
    plsc.subcore_barrier()

    def ablock(k, carry):
      blk = sid + k * NS
      par = lax.rem(k, 2)

      @pl.when(blk < nbt)
      def _():
        wait_stage(par)

        @pl.when(blk + NS < nbt)
        def _():
          fire_stage(blk + NS, 1 - par)

        @pl.when(k >= 2)
        def _():
          drain_ascatters(par)

        def rowloop(rr, c2):
          kr = par * CPB + rr
          for g in range(CH // 16):
            fb = rr * CH + g * 16
            sl = pl.ds(fb, 16)
            gs = pl.ds(g * 16, 16)
            key_a[kr, gs] = ets2[par, sl] * n + eits[par, 1, sl]
            w = (ef4s[par, 0, sl] + ef4s[par, 1, sl] + ef4s[par, 2, sl]
                 + ef4s[par, 3, sl])
            wabs_a[kr, gs] = jnp.abs(w * 0.25)
          return c2

        lax.fori_loop(0, CPB, rowloop, 0)
        for jj in range(CPB):
          pltpu.async_copy(wabs_a.at[par * CPB + jj],
                           deg_sp.at[key_a.at[par * CPB + jj]],
                           sem_a.at[par], add=True)

      return carry

    lax.fori_loop(0, nba, ablock, 0)
    for dk in (nba - 2, nba - 1):
      @pl.when(sid + dk * NS < nbt)
      def _(dk=dk):
        drain_ascatters(dk % 2)

    fire_stage(wid, 0)
    plsc.subcore_barrier()

    nw = jnp.where(wid < remw, nwmax, nwmax - CPB) if remw else nwmax

    def cpipe(j, carry):
      cc = lax.rem(j, CPB)
      kb = lax.div(j, CPB)
      par = lax.rem(j, 2)
      pb = lax.rem(kb, 2)

      @pl.when(jnp.logical_and(j >= 2, j - 2 < nw))
      def _():
        pltpu.make_async_copy(rows2.at[par], out_sp.at[srow2.at[par]],
                              sem_s.at[par]).wait()

      @pl.when(j < nw)
      def _():
        @pl.when(cc == 0)
        def _():
          wait_stage(pb)

          @pl.when((kb + 1) * CPB < nw)
          def _():
            fire_stage(wid + (kb + 1) * NW, 1 - pb)

        for g in range(CH // 16):
          fb = cc * CH + g * 16
          sl = pl.ds(fb, 16)
          gs = pl.ds(g * 16, 16)
          key_a[cc, gs] = ets2[pb, sl] * n + eits[pb, 1, sl]
          srow2[par, gs] = eits[pb, 0, sl]
          w = (ef4s[pb, 0, sl] + ef4s[pb, 1, sl] + ef4s[pb, 2, sl]
               + ef4s[pb, 3, sl])
          c2v[par, gs] = w * 0.25
        pltpu.sync_copy(deg_sp.at[key_a.at[cc]], degc)
        for g in range(CH // 16):
          gs = pl.ds(g * 16, 16)
          c2v[par, gs] = c2v[par, gs] / (degc[gs] + 1e-8)
        pltpu.async_copy(trel.at[key_a.at[cc]], rows2.at[par],
                         sem_g.at[par])

      @pl.when(jnp.logical_and(j >= 1, j - 1 < nw))
      def _():
        rp = lax.rem(j - 1, CPB)
        pq = 1 - par
        pltpu.make_async_copy(trel.at[key_a.at[rp]], rows2.at[pq],
                              sem_g.at[pq]).wait()

        def scale(g, c3):
          c16 = c2v[pq, pl.ds(g * 16, 16)]
          for k in range(16):
            spl = jnp.take(c16, jnp.full((16,), k, jnp.int32), mode="fill")
            erow = g * 16 + k
            for h in range(d // 16):
              sl = pl.ds(h * 16, 16)
              rows2[pq, erow, sl] = rows2[pq, erow, sl] * spl
          return c3

        lax.fori_loop(0, CH // 16, scale, 0)
        pltpu.async_copy(rows2.at[pq], out_sp.at[srow2.at[pq]],
                         sem_s.at[pq], add=True)

      return carry

    lax.fori_loop(0, nwmax + 2, cpipe, 0)
    plsc.subcore_barrier()

    def wb(k, carry):
      ch_id = sid + k * NS

      @pl.when(ch_id < nrc)
      def _():
        pltpu.sync_copy(out_sp.at[pl.ds(ch_id * WBC, WBC)],
                        rows2.at[0, pl.ds(0, WBC)])
        pltpu.sync_copy(rows2.at[0, pl.ds(0, WBC)],
                        out_hbm.at[cid, pl.ds(ch_id * WBC, WBC)])

      return carry

    lax.fori_loop(0, -(-nrc // NS), wb, 0)

  return sc_kernel


def kernel(node_features, edge_features, adj_e, adj_v, T, edge_index,
           edge_type, W_rel, W_self, b_self):
  n = adj_v.shape[0]
  e = edge_index.shape[1]
  din = node_features.shape[1]
  dout = W_self.shape[0]
  r = W_rel.shape[0]

  bm = 2000
  nb = n // bm
  trel = pl.pallas_call(
      _rel_transform_body,
      grid=(r, nb),
      in_specs=[
          pl.BlockSpec((bm, din), lambda q, i: (i, 0)),
          pl.BlockSpec((1, dout, din), lambda q, i: (q, 0, 0)),
      ],
      out_specs=pl.BlockSpec((bm, dout), lambda q, i: (q * nb + i, 0)),
      out_shape=jax.ShapeDtypeStruct((r * n, dout), jnp.float32),
  )(node_features, W_rel)

  self_out = pl.pallas_call(
      _self_transform_body,
      grid=(nb,),
      in_specs=[
          pl.BlockSpec((bm, din), lambda i: (i, 0)),
          pl.BlockSpec((dout, din), lambda i: (0, 0)),
          pl.BlockSpec((1, dout), lambda i: (0, 0)),
      ],
      out_specs=pl.BlockSpec((bm, dout), lambda i: (i, 0)),
      out_shape=jax.ShapeDtypeStruct((n, dout), jnp.float32),
  )(node_features, W_self, b_self.reshape(1, dout))

  ef_t = edge_features.T
  partials = _make_sc_kernel(n, e, dout, r)(
      trel, edge_type, edge_index, ef_t)

  out = pl.pallas_call(
      _combine_body,
      grid=(n // bm,),
      in_specs=[
          pl.BlockSpec((NC, bm, dout), lambda i: (0, i, 0)),
          pl.BlockSpec((bm, dout), lambda i: (i, 0)),
      ],
      out_specs=pl.BlockSpec((bm, dout), lambda i: (i, 0)),
      out_shape=jax.ShapeDtypeStruct((n, dout), jnp.float32),
  )(partials, self_out)

  return out, edge_features


# speedup vs baseline: 2.1369x; 2.1369x over previous
import functools

import jax
import jax.numpy as jnp
from jax import lax
from jax.experimental import pallas as pl
from jax.experimental.pallas import tpu as pltpu
from jax.experimental.pallas import tpu_sc as plsc

NC = 2
NS = 16
NW = NC * NS

CH = 128
SBLK = 512
CPB = SBLK // CH
WBC = 80


def _rel_transform_body(x_ref, w_ref, o_ref):
  o_ref[...] = lax.dot_general(
      x_ref[...], w_ref[0],
      dimension_numbers=(((1,), (1,)), ((), ())),
      preferred_element_type=jnp.float32,
  )


def _self_transform_body(x_ref, w_ref, b_ref, o_ref):
  o_ref[...] = lax.dot_general(
      x_ref[...], w_ref[...],
      dimension_numbers=(((1,), (1,)), ((), ())),
      preferred_element_type=jnp.float32,
  ) + b_ref[...]


def _combine_body(p_ref, s_ref, o_ref):
  o_ref[...] = p_ref[0] + p_ref[1] + s_ref[...]


def _make_sc_kernel(n, e, d, r):
  nbt = e // SBLK
  nba = -(-nbt // NS)
  nbw = -(-nbt // NW)
  remw = nbt % NW
  nwmax = nbw * CPB
  deg_sz = r * n
  nrc = n // WBC
  ndz = deg_sz // 2000

  mesh = plsc.VectorSubcoreMesh(core_axis_name="c", subcore_axis_name="s")

  @functools.partial(
      pl.kernel,
      mesh=mesh,
      compiler_params=pltpu.CompilerParams(needs_layout_passes=False),
      out_type=jax.ShapeDtypeStruct((NC, n, d), jnp.float32),
      scratch_types=[
          pltpu.VMEM_SHARED((deg_sz,), jnp.float32),
          pltpu.VMEM_SHARED((n, d), jnp.float32),
          pltpu.VMEM((4, SBLK), jnp.float32),
          pltpu.VMEM((2, SBLK), jnp.int32),
          pltpu.VMEM((SBLK,), jnp.int32),
          pltpu.VMEM((2 * CPB, CH), jnp.int32),
          pltpu.VMEM((2 * CPB, CH), jnp.float32),
          pltpu.VMEM((CH, d), jnp.float32),
          pltpu.VMEM((CH, d), jnp.float32),
          pltpu.VMEM((CH,), jnp.float32),
          pltpu.VMEM((CH,), jnp.float32),
          pltpu.VMEM((CH,), jnp.int32),
          pltpu.VMEM((CH,), jnp.int32),
          pltpu.VMEM((CH,), jnp.float32),
          pltpu.VMEM((2000,), jnp.float32),
          pltpu.SemaphoreType.DMA,
          pltpu.SemaphoreType.DMA,
          pltpu.SemaphoreType.DMA,
          pltpu.SemaphoreType.DMA,
          pltpu.SemaphoreType.DMA,
          pltpu.SemaphoreType.DMA,
          pltpu.SemaphoreType.DMA,
      ],
  )
  def sc_kernel(trel, et_h, ei_h, ef_t, out_hbm,
                deg_sp, out_sp,
                ef4s, eits, ets, key_a, wabs_a,
                rows0, rows1, c0, c1, srow0, srow1, degc, zbuf,
                sem_g0, sem_g1, sem_s0, sem_s1, sem_a0, sem_a1, sem_b):
    cid = lax.axis_index("c")
    sid = lax.axis_index("s")
    wid = sid * NC + cid
    z16 = jnp.zeros((16,), jnp.float32)

    def stage_block(off):
      d1 = pltpu.async_copy(ei_h.at[:, pl.ds(off, SBLK)], eits, sem_b)
      d2 = pltpu.async_copy(ef_t.at[:, pl.ds(off, SBLK)], ef4s, sem_b)
      d3 = pltpu.async_copy(et_h.at[pl.ds(off, SBLK)], ets, sem_b)
      d1.wait()
      d2.wait()
      d3.wait()

    def drain_ascatters(p, sem_ab):
      for jj in range(CPB):
        pltpu.make_async_copy(wabs_a.at[p * CPB + jj],
                              deg_sp.at[key_a.at[p * CPB + jj]],
                              sem_ab).wait()

    def zrow(i, carry):
      for h in range(d // 16):
        rows0[i, pl.ds(h * 16, 16)] = z16
      return carry

    lax.fori_loop(0, CH, zrow, 0)

    def zout(k, carry):
      ch_id = sid + k * NS

      @pl.when(ch_id < nrc)
      def _():
        pltpu.sync_copy(rows0.at[pl.ds(0, WBC)],
                        out_sp.at[pl.ds(ch_id * WBC, WBC)])

      return carry

    lax.fori_loop(0, -(-nrc // NS), zout, 0)

    def zc(i, carry):
      zbuf[pl.ds(i * 16, 16)] = z16
      return carry

    lax.fori_loop(0, 125, zc, 0)

    def zdeg(k, carry):
      ch_id = sid + k * NS

      @pl.when(ch_id < ndz)
      def _():
        pltpu.sync_copy(zbuf, deg_sp.at[pl.ds(ch_id * 2000, 2000)])

      return carry

    lax.fori_loop(0, -(-ndz // NS), zdeg, 0)
    plsc.subcore_barrier()

    def aproc(p, sem_ab, k):
      @pl.when(k >= 2)
      def _():
        drain_ascatters(p, sem_ab)

      def rowloop(rr, c2):
        kr = p * CPB + rr
        for g in range(CH // 16):
          fb = rr * CH + g * 16
          sl = pl.ds(fb, 16)
          gs = pl.ds(g * 16, 16)
          key_a[kr, gs] = ets[sl] * n + eits[1, sl]
          w = ef4s[0, sl] + ef4s[1, sl] + ef4s[2, sl] + ef4s[3, sl]
          wabs_a[kr, gs] = jnp.abs(w * 0.25)
        return c2

      lax.fori_loop(0, CPB, rowloop, 0)
      for jj in range(CPB):
        pltpu.async_copy(wabs_a.at[p * CPB + jj],
                         deg_sp.at[key_a.at[p * CPB + jj]],
                         sem_ab, add=True)

    def ablock(k, carry):
      blk = sid + k * NS

      @pl.when(blk < nbt)
      def _():
        stage_block(blk * SBLK)

        @pl.when(lax.rem(k, 2) == 0)
        def _():
          aproc(0, sem_a0, k)

        @pl.when(lax.rem(k, 2) == 1)
        def _():
          aproc(1, sem_a1, k)

      return carry

    lax.fori_loop(0, nba, ablock, 0)
    for dk in (nba - 2, nba - 1):
      @pl.when(sid + dk * NS < nbt)
      def _(dk=dk):
        drain_ascatters(dk % 2, sem_a0 if dk % 2 == 0 else sem_a1)

    plsc.subcore_barrier()

    nw = jnp.where(wid < remw, nwmax, nwmax - CPB) if remw else nwmax

    def cpipe(j, carry):
      cc = lax.rem(j, CPB)
      par = lax.rem(j, 2)

      @pl.when(jnp.logical_and(j >= 2, j - 2 < nw))
      def _():
        @pl.when(par == 0)
        def _():
          pltpu.make_async_copy(rows0, out_sp.at[srow0], sem_s0).wait()

        @pl.when(par == 1)
        def _():
          pltpu.make_async_copy(rows1, out_sp.at[srow1], sem_s1).wait()

      @pl.when(j < nw)
      def _():
        @pl.when(cc == 0)
        def _():
          stage_block((wid + lax.div(j, CPB) * NW) * SBLK)

        def front(rowsb, cb, srowb, sem_gb):
          for g in range(CH // 16):
            fb = cc * CH + g * 16
            sl = pl.ds(fb, 16)
            gs = pl.ds(g * 16, 16)
            key_a[cc, gs] = ets[sl] * n + eits[1, sl]
            srowb[gs] = eits[0, sl]
            w = ef4s[0, sl] + ef4s[1, sl] + ef4s[2, sl] + ef4s[3, sl]
            cb[gs] = w * 0.25
          pltpu.sync_copy(deg_sp.at[key_a.at[cc]], degc)
          for g in range(CH // 16):
            gs = pl.ds(g * 16, 16)
            cb[gs] = cb[gs] / (degc[gs] + 1e-8)
          pltpu.async_copy(trel.at[key_a.at[cc]], rowsb, sem_gb)

        @pl.when(par == 0)
        def _():
          front(rows0, c0, srow0, sem_g0)

        @pl.when(par == 1)
        def _():
          front(rows1, c1, srow1, sem_g1)

      @pl.when(jnp.logical_and(j >= 1, j - 1 < nw))
      def _():
        rp = lax.rem(j - 1, CPB)

        def back(rowsb, cb, srowb, sem_gb, sem_sb):
          pltpu.make_async_copy(trel.at[key_a.at[rp]], rowsb, sem_gb).wait()

          def scale(g, c3):
            c16 = cb[pl.ds(g * 16, 16)]
            for k in range(16):
              spl = jnp.take(c16, jnp.full((16,), k, jnp.int32), mode="fill")
              erow = g * 16 + k
              for h in range(d // 16):
                sl = pl.ds(h * 16, 16)
                rowsb[erow, sl] = rowsb[erow, sl] * spl
            return c3

          lax.fori_loop(0, CH // 16, scale, 0)
          pltpu.async_copy(rowsb, out_sp.at[srowb], sem_sb, add=True)

        @pl.when(par == 1)
        def _():
          back(rows0, c0, srow0, sem_g0, sem_s0)

        @pl.when(par == 0)
        def _():
          back(rows1, c1, srow1, sem_g1, sem_s1)

      return carry

    lax.fori_loop(0, nwmax + 2, cpipe, 0)
    plsc.subcore_barrier()

    def wb(k, carry):
      ch_id = sid + k * NS

      @pl.when(ch_id < nrc)
      def _():
        pltpu.sync_copy(out_sp.at[pl.ds(ch_id * WBC, WBC)],
                        rows0.at[pl.ds(0, WBC)])
        pltpu.sync_copy(rows0.at[pl.ds(0, WBC)],
                        out_hbm.at[cid, pl.ds(ch_id * WBC, WBC)])

      return carry

    lax.fori_loop(0, -(-nrc // NS), wb, 0)

  return sc_kernel


def kernel(node_features, edge_features, adj_e, adj_v, T, edge_index,
           edge_type, W_rel, W_self, b_self):
  n = adj_v.shape[0]
  e = edge_index.shape[1]
  din = node_features.shape[1]
  dout = W_self.shape[0]
  r = W_rel.shape[0]

  bm = 2000
  nb = n // bm
  trel = pl.pallas_call(
      _rel_transform_body,
      grid=(r, nb),
      in_specs=[
          pl.BlockSpec((bm, din), lambda q, i: (i, 0)),
          pl.BlockSpec((1, dout, din), lambda q, i: (q, 0, 0)),
      ],
      out_specs=pl.BlockSpec((bm, dout), lambda q, i: (q * nb + i, 0)),
      out_shape=jax.ShapeDtypeStruct((r * n, dout), jnp.float32),
  )(node_features, W_rel)

  self_out = pl.pallas_call(
      _self_transform_body,
      grid=(nb,),
      in_specs=[
          pl.BlockSpec((bm, din), lambda i: (i, 0)),
          pl.BlockSpec((dout, din), lambda i: (0, 0)),
          pl.BlockSpec((1, dout), lambda i: (0, 0)),
      ],
      out_specs=pl.BlockSpec((bm, dout), lambda i: (i, 0)),
      out_shape=jax.ShapeDtypeStruct((n, dout), jnp.float32),
  )(node_features, W_self, b_self.reshape(1, dout))

  ef_t = edge_features.T
  partials = _make_sc_kernel(n, e, dout, r)(
      trel, edge_type, edge_index, ef_t)

  out = pl.pallas_call(
      _combine_body,
      grid=(n // bm,),
      in_specs=[
          pl.BlockSpec((NC, bm, dout), lambda i: (0, i, 0)),
          pl.BlockSpec((bm, dout), lambda i: (i, 0)),
      ],
      out_specs=pl.BlockSpec((bm, dout), lambda i: (i, 0)),
      out_shape=jax.ShapeDtypeStruct((n, dout), jnp.float32),
  )(partials, self_out)

  return out, edge_features


# speedup vs baseline: 2.2210x; 1.0394x over previous
import functools

import jax
import jax.numpy as jnp
from jax import lax
from jax.experimental import pallas as pl
from jax.experimental.pallas import tpu as pltpu
from jax.experimental.pallas import tpu_sc as plsc

NC = 2
NS = 16
NW = NC * NS

CH = 128
SBLK = 512
CPB = SBLK // CH
WBC = 80


def _rel_transform_body(x_ref, w_ref, o_ref):
  o_ref[...] = lax.dot_general(
      x_ref[...], w_ref[0],
      dimension_numbers=(((1,), (1,)), ((), ())),
      preferred_element_type=jnp.float32,
  )


def _self_transform_body(x_ref, w_ref, b_ref, o_ref):
  o_ref[...] = lax.dot_general(
      x_ref[...], w_ref[...],
      dimension_numbers=(((1,), (1,)), ((), ())),
      preferred_element_type=jnp.float32,
  ) + b_ref[...]


def _combine_body(p_ref, s_ref, o_ref):
  o_ref[...] = p_ref[0] + p_ref[1] + s_ref[...]


def _make_sc_kernel(n, e, d, r):
  nbt = e // SBLK
  nba = -(-nbt // NS)
  nbw = -(-nbt // NW)
  remw = nbt % NW
  nwmax = nbw * CPB
  deg_sz = r * n
  nrc = n // WBC
  ndz = deg_sz // 2000

  mesh = plsc.VectorSubcoreMesh(core_axis_name="c", subcore_axis_name="s")

  @functools.partial(
      pl.kernel,
      mesh=mesh,
      compiler_params=pltpu.CompilerParams(needs_layout_passes=False),
      out_type=jax.ShapeDtypeStruct((NC, n, d), jnp.float32),
      scratch_types=[
          pltpu.VMEM_SHARED((deg_sz,), jnp.float32),
          pltpu.VMEM_SHARED((n, d), jnp.float32),
          pltpu.VMEM((4, SBLK), jnp.float32),
          pltpu.VMEM((2, SBLK), jnp.int32),
          pltpu.VMEM((SBLK,), jnp.int32),
          pltpu.VMEM((2 * CPB, CH), jnp.int32),
          pltpu.VMEM((2 * CPB, CH), jnp.float32),
          pltpu.VMEM((CH, d), jnp.float32),
          pltpu.VMEM((CH, d), jnp.float32),
          pltpu.VMEM((CH,), jnp.float32),
          pltpu.VMEM((CH,), jnp.float32),
          pltpu.VMEM((CH,), jnp.int32),
          pltpu.VMEM((CH,), jnp.int32),
          pltpu.VMEM((CH,), jnp.float32),
          pltpu.VMEM((CH,), jnp.float32),
          pltpu.VMEM((2000,), jnp.float32),
          pltpu.SemaphoreType.DMA,
          pltpu.SemaphoreType.DMA,
          pltpu.SemaphoreType.DMA,
          pltpu.SemaphoreType.DMA,
          pltpu.SemaphoreType.DMA,
          pltpu.SemaphoreType.DMA,
          pltpu.SemaphoreType.DMA,
          pltpu.SemaphoreType.DMA,
          pltpu.SemaphoreType.DMA,
      ],
  )
  def sc_kernel(trel, et_h, ei_h, ef_t, out_hbm,
                deg_sp, out_sp,
                ef4s, eits, ets, key_a, wabs_a,
                rows0, rows1, c0, c1, srow0, srow1, degc0, degc1, zbuf,
                sem_g0, sem_g1, sem_s0, sem_s1, sem_a0, sem_a1, sem_b,
                sem_d0, sem_d1):
    cid = lax.axis_index("c")
    sid = lax.axis_index("s")
    wid = sid * NC + cid
    z16 = jnp.zeros((16,), jnp.float32)

    def stage_block(off):
      d1 = pltpu.async_copy(ei_h.at[:, pl.ds(off, SBLK)], eits, sem_b)
      d2 = pltpu.async_copy(ef_t.at[:, pl.ds(off, SBLK)], ef4s, sem_b)
      d3 = pltpu.async_copy(et_h.at[pl.ds(off, SBLK)], ets, sem_b)
      d1.wait()
      d2.wait()
      d3.wait()

    def drain_ascatters(p, sem_ab):
      for jj in range(CPB):
        pltpu.make_async_copy(wabs_a.at[p * CPB + jj],
                              deg_sp.at[key_a.at[p * CPB + jj]],
                              sem_ab).wait()

    def zrow(i, carry):
      for h in range(d // 16):
        rows0[i, pl.ds(h * 16, 16)] = z16
      return carry

    lax.fori_loop(0, CH, zrow, 0)

    def zout(k, carry):
      ch_id = sid + k * NS

      @pl.when(ch_id < nrc)
      def _():
        pltpu.sync_copy(rows0.at[pl.ds(0, WBC)],
                        out_sp.at[pl.ds(ch_id * WBC, WBC)])

      return carry

    lax.fori_loop(0, -(-nrc // NS), zout, 0)

    def zc(i, carry):
      zbuf[pl.ds(i * 16, 16)] = z16
      return carry

    lax.fori_loop(0, 125, zc, 0)

    def zdeg(k, carry):
      ch_id = sid + k * NS

      @pl.when(ch_id < ndz)
      def _():
        pltpu.sync_copy(zbuf, deg_sp.at[pl.ds(ch_id * 2000, 2000)])

      return carry

    lax.fori_loop(0, -(-ndz // NS), zdeg, 0)
    plsc.subcore_barrier()

    def aproc(p, sem_ab, k):
      @pl.when(k >= 2)
      def _():
        drain_ascatters(p, sem_ab)

      def rowloop(rr, c2):
        kr = p * CPB + rr
        for g in range(CH // 16):
          fb = rr * CH + g * 16
          sl = pl.ds(fb, 16)
          gs = pl.ds(g * 16, 16)
          key_a[kr, gs] = ets[sl] * n + eits[1, sl]
          w = ef4s[0, sl] + ef4s[1, sl] + ef4s[2, sl] + ef4s[3, sl]
          wabs_a[kr, gs] = jnp.abs(w * 0.25)
        return c2

      lax.fori_loop(0, CPB, rowloop, 0)
      for jj in range(CPB):
        pltpu.async_copy(wabs_a.at[p * CPB + jj],
                         deg_sp.at[key_a.at[p * CPB + jj]],
                         sem_ab, add=True)

    def ablock(k, carry):
      blk = sid + k * NS

      @pl.when(blk < nbt)
      def _():
        stage_block(blk * SBLK)

        @pl.when(lax.rem(k, 2) == 0)
        def _():
          aproc(0, sem_a0, k)

        @pl.when(lax.rem(k, 2) == 1)
        def _():
          aproc(1, sem_a1, k)

      return carry

    lax.fori_loop(0, nba, ablock, 0)
    for dk in (nba - 2, nba - 1):
      @pl.when(sid + dk * NS < nbt)
      def _(dk=dk):
        drain_ascatters(dk % 2, sem_a0 if dk % 2 == 0 else sem_a1)

    plsc.subcore_barrier()

    nw = jnp.where(wid < remw, nwmax, nwmax - CPB) if remw else nwmax

    def cpipe(j, carry):
      cc = lax.rem(j, CPB)
      par = lax.rem(j, 2)

      @pl.when(jnp.logical_and(j >= 2, j - 2 < nw))
      def _():
        @pl.when(par == 0)
        def _():
          pltpu.make_async_copy(rows0, out_sp.at[srow0], sem_s0).wait()

        @pl.when(par == 1)
        def _():
          pltpu.make_async_copy(rows1, out_sp.at[srow1], sem_s1).wait()

      @pl.when(j < nw)
      def _():
        @pl.when(cc == 0)
        def _():
          stage_block((wid + lax.div(j, CPB) * NW) * SBLK)

        def front(rowsb, cb, srowb, degb, sem_gb, sem_db):
          for g in range(CH // 16):
            fb = cc * CH + g * 16
            sl = pl.ds(fb, 16)
            gs = pl.ds(g * 16, 16)
            key_a[cc, gs] = ets[sl] * n + eits[1, sl]
            srowb[gs] = eits[0, sl]
            w = ef4s[0, sl] + ef4s[1, sl] + ef4s[2, sl] + ef4s[3, sl]
            cb[gs] = w * 0.25
          pltpu.async_copy(deg_sp.at[key_a.at[cc]], degb, sem_db)
          pltpu.async_copy(trel.at[key_a.at[cc]], rowsb, sem_gb)

        @pl.when(par == 0)
        def _():
          front(rows0, c0, srow0, degc0, sem_g0, sem_d0)

        @pl.when(par == 1)
        def _():
          front(rows1, c1, srow1, degc1, sem_g1, sem_d1)

      @pl.when(jnp.logical_and(j >= 1, j - 1 < nw))
      def _():
        rp = lax.rem(j - 1, CPB)

        def back(rowsb, cb, srowb, degb, sem_gb, sem_sb, sem_db):
          pltpu.make_async_copy(deg_sp.at[key_a.at[rp]], degb, sem_db).wait()
          pltpu.make_async_copy(trel.at[key_a.at[rp]], rowsb, sem_gb).wait()
          for g in range(CH // 16):
            gs = pl.ds(g * 16, 16)
            cb[gs] = cb[gs] / (degb[gs] + 1e-8)

          def scale(g, c3):
            c16 = cb[pl.ds(g * 16, 16)]
            for k in range(16):
              spl = jnp.take(c16, jnp.full((16,), k, jnp.int32), mode="fill")
              erow = g * 16 + k
              for h in range(d // 16):
                sl = pl.ds(h * 16, 16)
                rowsb[erow, sl] = rowsb[erow, sl] * spl
            return c3

          lax.fori_loop(0, CH // 16, scale, 0)
          pltpu.async_copy(rowsb, out_sp.at[srowb], sem_sb, add=True)

        @pl.when(par == 1)
        def _():
          back(rows0, c0, srow0, degc0, sem_g0, sem_s0, sem_d0)

        @pl.when(par == 0)
        def _():
          back(rows1, c1, srow1, degc1, sem_g1, sem_s1, sem_d1)

      return carry

    lax.fori_loop(0, nwmax + 2, cpipe, 0)
    plsc.subcore_barrier()

    def wb(k, carry):
      ch_id = sid + k * NS

      @pl.when(ch_id < nrc)
      def _():
        pltpu.sync_copy(out_sp.at[pl.ds(ch_id * WBC, WBC)],
                        rows0.at[pl.ds(0, WBC)])
        pltpu.sync_copy(rows0.at[pl.ds(0, WBC)],
                        out_hbm.at[cid, pl.ds(ch_id * WBC, WBC)])

      return carry

    lax.fori_loop(0, -(-nrc // NS), wb, 0)

  return sc_kernel


def kernel(node_features, edge_features, adj_e, adj_v, T, edge_index,
           edge_type, W_rel, W_self, b_self):
  n = adj_v.shape[0]
  e = edge_index.shape[1]
  din = node_features.shape[1]
  dout = W_self.shape[0]
  r = W_rel.shape[0]

  bm = 2000
  nb = n // bm
  trel = pl.pallas_call(
      _rel_transform_body,
      grid=(r, nb),
      in_specs=[
          pl.BlockSpec((bm, din), lambda q, i: (i, 0)),
          pl.BlockSpec((1, dout, din), lambda q, i: (q, 0, 0)),
      ],
      out_specs=pl.BlockSpec((bm, dout), lambda q, i: (q * nb + i, 0)),
      out_shape=jax.ShapeDtypeStruct((r * n, dout), jnp.float32),
  )(node_features, W_rel)

  self_out = pl.pallas_call(
      _self_transform_body,
      grid=(nb,),
      in_specs=[
          pl.BlockSpec((bm, din), lambda i: (i, 0)),
          pl.BlockSpec((dout, din), lambda i: (0, 0)),
          pl.BlockSpec((1, dout), lambda i: (0, 0)),
      ],
      out_specs=pl.BlockSpec((bm, dout), lambda i: (i, 0)),
      out_shape=jax.ShapeDtypeStruct((n, dout), jnp.float32),
  )(node_features, W_self, b_self.reshape(1, dout))

  ef_t = edge_features.T
  partials = _make_sc_kernel(n, e, dout, r)(
      trel, edge_type, edge_index, ef_t)

  out = pl.pallas_call(
      _combine_body,
      grid=(n // bm,),
      in_specs=[
          pl.BlockSpec((NC, bm, dout), lambda i: (0, i, 0)),
          pl.BlockSpec((bm, dout), lambda i: (i, 0)),
      ],
      out_specs=pl.BlockSpec((bm, dout), lambda i: (i, 0)),
      out_shape=jax.ShapeDtypeStruct((n, dout), jnp.float32),
  )(partials, self_out)

  return out, edge_features


# speedup vs baseline: 2.3847x; 1.0737x over previous
import functools

import jax
import jax.numpy as jnp
from jax import lax
from jax.experimental import pallas as pl
from jax.experimental.pallas import tpu as pltpu
from jax.experimental.pallas import tpu_sc as plsc

NC = 2
NS = 16
NW = NC * NS

CH = 128
SBLK = 512
CPB = SBLK // CH
WBC = 80


def _rel_transform_body(x_ref, w_ref, o_ref):
  o_ref[...] = lax.dot_general(
      x_ref[...], w_ref[0],
      dimension_numbers=(((1,), (1,)), ((), ())),
      preferred_element_type=jnp.float32,
  )


def _self_transform_body(x_ref, w_ref, b_ref, o_ref):
  o_ref[...] = lax.dot_general(
      x_ref[...], w_ref[...],
      dimension_numbers=(((1,), (1,)), ((), ())),
      preferred_element_type=jnp.float32,
  ) + b_ref[...]


def _combine_body(p_ref, s_ref, o_ref):
  o_ref[...] = p_ref[0] + p_ref[1] + s_ref[...]


def _make_sc_kernel(n, e, d, r):
  nbt = e // SBLK
  nba = -(-nbt // NS)
  nbw = -(-nbt // NW)
  remw = nbt % NW
  nwmax = nbw * CPB
  deg_sz = r * n
  nrc = n // WBC
  ndz = deg_sz // 2000

  mesh = plsc.VectorSubcoreMesh(core_axis_name="c", subcore_axis_name="s")

  @functools.partial(
      pl.kernel,
      mesh=mesh,
      compiler_params=pltpu.CompilerParams(needs_layout_passes=False),
      out_type=jax.ShapeDtypeStruct((NC, n, d), jnp.float32),
      scratch_types=[
          pltpu.VMEM_SHARED((deg_sz,), jnp.float32),
          pltpu.VMEM_SHARED((n, d), jnp.float32),
          pltpu.VMEM((2, 4, SBLK), jnp.float32),
          pltpu.VMEM((2, 2, SBLK), jnp.int32),
          pltpu.VMEM((2, SBLK), jnp.int32),
          pltpu.VMEM((2 * CPB, CH), jnp.int32),
          pltpu.VMEM((2 * CPB, CH), jnp.float32),
          pltpu.VMEM((CH, d), jnp.float32),
          pltpu.VMEM((CH, d), jnp.float32),
          pltpu.VMEM((CH,), jnp.float32),
          pltpu.VMEM((CH,), jnp.float32),
          pltpu.VMEM((CH,), jnp.int32),
          pltpu.VMEM((CH,), jnp.int32),
          pltpu.VMEM((CH,), jnp.float32),
          pltpu.VMEM((CH,), jnp.float32),
          pltpu.VMEM((2000,), jnp.float32),
          pltpu.SemaphoreType.DMA,
          pltpu.SemaphoreType.DMA,
          pltpu.SemaphoreType.DMA,
          pltpu.SemaphoreType.DMA,
          pltpu.SemaphoreType.DMA,
          pltpu.SemaphoreType.DMA,
          pltpu.SemaphoreType.DMA,
          pltpu.SemaphoreType.DMA,
          pltpu.SemaphoreType.DMA,
      ],
  )
  def sc_kernel(trel, et_h, ei_h, ef_t, out_hbm,
                deg_sp, out_sp,
                ef4s, eits, ets, key_a, wabs_a,
                rows0, rows1, c0, c1, srow0, srow1, degc0, degc1, zbuf,
                sem_g0, sem_g1, sem_s0, sem_s1, sem_a0, sem_a1, sem_b,
                sem_d0, sem_d1):
    cid = lax.axis_index("c")
    sid = lax.axis_index("s")
    wid = sid * NC + cid
    z16 = jnp.zeros((16,), jnp.float32)

    def fire_stage(off, p):
      pltpu.async_copy(ei_h.at[:, pl.ds(off, SBLK)], eits.at[p], sem_b)
      pltpu.async_copy(ef_t.at[:, pl.ds(off, SBLK)], ef4s.at[p], sem_b)
      pltpu.async_copy(et_h.at[pl.ds(off, SBLK)], ets.at[p], sem_b)

    def wait_stage(p):
      pltpu.make_async_copy(ei_h.at[:, pl.ds(0, SBLK)], eits.at[p],
                            sem_b).wait()
      pltpu.make_async_copy(ef_t.at[:, pl.ds(0, SBLK)], ef4s.at[p],
                            sem_b).wait()
      pltpu.make_async_copy(et_h.at[pl.ds(0, SBLK)], ets.at[p],
                            sem_b).wait()

    def stage_block(off):
      fire_stage(off, 0)
      wait_stage(0)

    def drain_ascatters(p, sem_ab):
      for jj in range(CPB):
        pltpu.make_async_copy(wabs_a.at[p * CPB + jj],
                              deg_sp.at[key_a.at[p * CPB + jj]],
                              sem_ab).wait()

    def zrow(i, carry):
      for h in range(d // 16):
        rows0[i, pl.ds(h * 16, 16)] = z16
      return carry

    lax.fori_loop(0, CH, zrow, 0)

    def zout(k, carry):
      ch_id = sid + k * NS

      @pl.when(ch_id < nrc)
      def _():
        pltpu.sync_copy(rows0.at[pl.ds(0, WBC)],
                        out_sp.at[pl.ds(ch_id * WBC, WBC)])

      return carry

    lax.fori_loop(0, -(-nrc // NS), zout, 0)

    def zc(i, carry):
      zbuf[pl.ds(i * 16, 16)] = z16
      return carry

    lax.fori_loop(0, 125, zc, 0)

    def zdeg(k, carry):
      ch_id = sid + k * NS

      @pl.when(ch_id < ndz)
      def _():
        pltpu.sync_copy(zbuf, deg_sp.at[pl.ds(ch_id * 2000, 2000)])

      return carry

    lax.fori_loop(0, -(-ndz // NS), zdeg, 0)
    plsc.subcore_barrier()

    def aproc(p, sem_ab, k, blk):
      wait_stage(p)

      @pl.when(blk + NS < nbt)
      def _():
        fire_stage((blk + NS) * SBLK, 1 - p)

      @pl.when(k >= 2)
      def _():
        drain_ascatters(p, sem_ab)

      def rowloop(rr, c2):
        kr = p * CPB + rr
        for g in range(CH // 16):
          fb = rr * CH + g * 16
          sl = pl.ds(fb, 16)
          gs = pl.ds(g * 16, 16)
          key_a[kr, gs] = ets[p, sl] * n + eits[p, 1, sl]
          w = (ef4s[p, 0, sl] + ef4s[p, 1, sl] + ef4s[p, 2, sl]
               + ef4s[p, 3, sl])
          wabs_a[kr, gs] = jnp.abs(w * 0.25)
        return c2

      lax.fori_loop(0, CPB, rowloop, 0)
      for jj in range(CPB):
        pltpu.async_copy(wabs_a.at[p * CPB + jj],
                         deg_sp.at[key_a.at[p * CPB + jj]],
                         sem_ab, add=True)

    fire_stage(sid * SBLK, 0)

    def ablock(k, carry):
      blk = sid + k * NS

      @pl.when(blk < nbt)
      def _():
        @pl.when(lax.rem(k, 2) == 0)
        def _():
          aproc(0, sem_a0, k, blk)

        @pl.when(lax.rem(k, 2) == 1)
        def _():
          aproc(1, sem_a1, k, blk)

      return carry

    lax.fori_loop(0, nba, ablock, 0)
    for dk in (nba - 2, nba - 1):
      @pl.when(sid + dk * NS < nbt)
      def _(dk=dk):
        drain_ascatters(dk % 2, sem_a0 if dk % 2 == 0 else sem_a1)

    plsc.subcore_barrier()

    nw = jnp.where(wid < remw, nwmax, nwmax - CPB) if remw else nwmax

    def cpipe(j, carry):
      cc = lax.rem(j, CPB)
      par = lax.rem(j, 2)

      @pl.when(jnp.logical_and(j >= 2, j - 2 < nw))
      def _():
        @pl.when(par == 0)
        def _():
          pltpu.make_async_copy(rows0, out_sp.at[srow0], sem_s0).wait()

        @pl.when(par == 1)
        def _():
          pltpu.make_async_copy(rows1, out_sp.at[srow1], sem_s1).wait()

      @pl.when(j < nw)
      def _():
        @pl.when(cc == 0)
        def _():
          stage_block((wid + lax.div(j, CPB) * NW) * SBLK)

        def front(rowsb, cb, srowb, degb, sem_gb, sem_db):
          for g in range(CH // 16):
            fb = cc * CH + g * 16
            sl = pl.ds(fb, 16)
            gs = pl.ds(g * 16, 16)
            key_a[cc, gs] = ets[0, sl] * n + eits[0, 1, sl]
            srowb[gs] = eits[0, 0, sl]
            w = (ef4s[0, 0, sl] + ef4s[0, 1, sl] + ef4s[0, 2, sl]
                 + ef4s[0, 3, sl])
            cb[gs] = w * 0.25
          pltpu.async_copy(deg_sp.at[key_a.at[cc]], degb, sem_db)
          pltpu.async_copy(trel.at[key_a.at[cc]], rowsb, sem_gb)

        @pl.when(par == 0)
        def _():
          front(rows0, c0, srow0, degc0, sem_g0, sem_d0)

        @pl.when(par == 1)
        def _():
          front(rows1, c1, srow1, degc1, sem_g1, sem_d1)

      @pl.when(jnp.logical_and(j >= 1, j - 1 < nw))
      def _():
        rp = lax.rem(j - 1, CPB)

        def back(rowsb, cb, srowb, degb, sem_gb, sem_sb, sem_db):
          pltpu.make_async_copy(deg_sp.at[key_a.at[rp]], degb, sem_db).wait()
          pltpu.make_async_copy(trel.at[key_a.at[rp]], rowsb, sem_gb).wait()
          for g in range(CH // 16):
            gs = pl.ds(g * 16, 16)
            cb[gs] = cb[gs] / (degb[gs] + 1e-8)

          def scale(g, c3):
            c16 = cb[pl.ds(g * 16, 16)]
            for k in range(16):
              spl = jnp.take(c16, jnp.full((16,), k, jnp.int32), mode="fill")
              erow = g * 16 + k
              for h in range(d // 16):
                sl = pl.ds(h * 16, 16)
                rowsb[erow, sl] = rowsb[erow, sl] * spl
            return c3

          lax.fori_loop(0, CH // 16, scale, 0)
          pltpu.async_copy(rowsb, out_sp.at[srowb], sem_sb, add=True)

        @pl.when(par == 1)
        def _():
          back(rows0, c0, srow0, degc0, sem_g0, sem_s0, sem_d0)

        @pl.when(par == 0)
        def _():
          back(rows1, c1, srow1, degc1, sem_g1, sem_s1, sem_d1)

      return carry

    lax.fori_loop(0, nwmax + 2, cpipe, 0)
    plsc.subcore_barrier()

    def wb(k, carry):
      ch_id = sid + k * NS

      @pl.when(ch_id < nrc)
      def _():
        pltpu.sync_copy(out_sp.at[pl.ds(ch_id * WBC, WBC)],
                        rows0.at[pl.ds(0, WBC)])
        pltpu.sync_copy(rows0.at[pl.ds(0, WBC)],
                        out_hbm.at[cid, pl.ds(ch_id * WBC, WBC)])

      return carry

    lax.fori_loop(0, -(-nrc // NS), wb, 0)

  return sc_kernel


def kernel(node_features, edge_features, adj_e, adj_v, T, edge_index,
           edge_type, W_rel, W_self, b_self):
  n = adj_v.shape[0]
  e = edge_index.shape[1]
  din = node_features.shape[1]
  dout = W_self.shape[0]
  r = W_rel.shape[0]

  bm = 2000
  nb = n // bm
  trel = pl.pallas_call(
      _rel_transform_body,
      grid=(r, nb),
      in_specs=[
          pl.BlockSpec((bm, din), lambda q, i: (i, 0)),
          pl.BlockSpec((1, dout, din), lambda q, i: (q, 0, 0)),
      ],
      out_specs=pl.BlockSpec((bm, dout), lambda q, i: (q * nb + i, 0)),
      out_shape=jax.ShapeDtypeStruct((r * n, dout), jnp.float32),
  )(node_features, W_rel)

  self_out = pl.pallas_call(
      _self_transform_body,
      grid=(nb,),
      in_specs=[
          pl.BlockSpec((bm, din), lambda i: (i, 0)),
          pl.BlockSpec((dout, din), lambda i: (0, 0)),
          pl.BlockSpec((1, dout), lambda i: (0, 0)),
      ],
      out_specs=pl.BlockSpec((bm, dout), lambda i: (i, 0)),
      out_shape=jax.ShapeDtypeStruct((n, dout), jnp.float32),
  )(node_features, W_self, b_self.reshape(1, dout))

  ef_t = edge_features.T
  partials = _make_sc_kernel(n, e, dout, r)(
      trel, edge_type, edge_index, ef_t)

  out = pl.pallas_call(
      _combine_body,
      grid=(n // bm,),
      in_specs=[
          pl.BlockSpec((NC, bm, dout), lambda i: (0, i, 0)),
          pl.BlockSpec((bm, dout), lambda i: (i, 0)),
      ],
      out_specs=pl.BlockSpec((bm, dout), lambda i: (i, 0)),
      out_shape=jax.ShapeDtypeStruct((n, dout), jnp.float32),
  )(partials, self_out)

  return out, edge_features


# speedup vs baseline: 2.5185x; 1.0561x over previous
import functools

import jax
import jax.numpy as jnp
from jax import lax
from jax.experimental import pallas as pl
from jax.experimental.pallas import tpu as pltpu
from jax.experimental.pallas import tpu_sc as plsc

NC = 2
NS = 16
NW = NC * NS

CH = 128
SBLK = 512
CPB = SBLK // CH
WBC = 80


def _rel_transform_body(x_ref, w_ref, o_ref):
  o_ref[...] = lax.dot_general(
      x_ref[...], w_ref[0],
      dimension_numbers=(((1,), (1,)), ((), ())),
      preferred_element_type=jnp.float32,
  )


def _self_transform_body(x_ref, w_ref, b_ref, o_ref):
  o_ref[...] = lax.dot_general(
      x_ref[...], w_ref[...],
      dimension_numbers=(((1,), (1,)), ((), ())),
      preferred_element_type=jnp.float32,
  ) + b_ref[...]


def _combine_body(p_ref, s_ref, o_ref):
  o_ref[...] = p_ref[0] + p_ref[1] + s_ref[...]


def _make_sc_kernel(n, e, d, r):
  nbt = e // SBLK
  nba = -(-nbt // NS)
  nbw = -(-nbt // NW)
  remw = nbt % NW
  nwmax = nbw * CPB
  deg_sz = r * n
  nrc = n // WBC
  ndz = deg_sz // 2000

  mesh = plsc.VectorSubcoreMesh(core_axis_name="c", subcore_axis_name="s")

  @functools.partial(
      pl.kernel,
      mesh=mesh,
      compiler_params=pltpu.CompilerParams(needs_layout_passes=False),
      out_type=jax.ShapeDtypeStruct((NC, n, d), jnp.float32),
      scratch_types=[
          pltpu.VMEM_SHARED((deg_sz,), jnp.float32),
          pltpu.VMEM_SHARED((n, d), jnp.float32),
          pltpu.VMEM((2, 4, SBLK), jnp.float32),
          pltpu.VMEM((2, 2, SBLK), jnp.int32),
          pltpu.VMEM((2, SBLK), jnp.int32),
          pltpu.VMEM((2 * CPB, CH), jnp.int32),
          pltpu.VMEM((2 * CPB, CH), jnp.float32),
          pltpu.VMEM((CH, d), jnp.float32),
          pltpu.VMEM((CH, d), jnp.float32),
          pltpu.VMEM((CH,), jnp.float32),
          pltpu.VMEM((CH,), jnp.float32),
          pltpu.VMEM((CH,), jnp.int32),
          pltpu.VMEM((CH,), jnp.int32),
          pltpu.VMEM((CH,), jnp.float32),
          pltpu.VMEM((CH,), jnp.float32),
          pltpu.VMEM((2000,), jnp.float32),
          pltpu.SemaphoreType.DMA,
          pltpu.SemaphoreType.DMA,
          pltpu.SemaphoreType.DMA,
          pltpu.SemaphoreType.DMA,
          pltpu.SemaphoreType.DMA,
          pltpu.SemaphoreType.DMA,
          pltpu.SemaphoreType.DMA,
          pltpu.SemaphoreType.DMA,
          pltpu.SemaphoreType.DMA,
      ],
  )
  def sc_kernel(trel, et_h, ei_h, ef_t, out_hbm,
                deg_sp, out_sp,
                ef4s, eits, ets, key_a, wabs_a,
                rows0, rows1, c0, c1, srow0, srow1, degc0, degc1, zbuf,
                sem_g0, sem_g1, sem_s0, sem_s1, sem_a0, sem_a1, sem_b,
                sem_d0, sem_d1):
    cid = lax.axis_index("c")
    sid = lax.axis_index("s")
    wid = sid * NC + cid
    z16 = jnp.zeros((16,), jnp.float32)

    def fire_stage(off, p):
      pltpu.async_copy(ei_h.at[:, pl.ds(off, SBLK)], eits.at[p], sem_b)
      pltpu.async_copy(ef_t.at[:, pl.ds(off, SBLK)], ef4s.at[p], sem_b)
      pltpu.async_copy(et_h.at[pl.ds(off, SBLK)], ets.at[p], sem_b)

    def wait_stage(p):
      pltpu.make_async_copy(ei_h.at[:, pl.ds(0, SBLK)], eits.at[p],
                            sem_b).wait()
      pltpu.make_async_copy(ef_t.at[:, pl.ds(0, SBLK)], ef4s.at[p],
                            sem_b).wait()
      pltpu.make_async_copy(et_h.at[pl.ds(0, SBLK)], ets.at[p],
                            sem_b).wait()

    def stage_block(off):
      fire_stage(off, 0)
      wait_stage(0)

    def drain_ascatters(p, sem_ab):
      for jj in range(CPB):
        pltpu.make_async_copy(wabs_a.at[p * CPB + jj],
                              deg_sp.at[key_a.at[p * CPB + jj]],
                              sem_ab).wait()

    def zrow(i, carry):
      for h in range(d // 16):
        rows0[i, pl.ds(h * 16, 16)] = z16
      return carry

    lax.fori_loop(0, CH, zrow, 0)

    def zout(k, carry):
      ch_id = sid + k * NS

      @pl.when(ch_id < nrc)
      def _():
        pltpu.sync_copy(rows0.at[pl.ds(0, WBC)],
                        out_sp.at[pl.ds(ch_id * WBC, WBC)])

      return carry

    lax.fori_loop(0, -(-nrc // NS), zout, 0)

    def zc(i, carry):
      zbuf[pl.ds(i * 16, 16)] = z16
      return carry

    lax.fori_loop(0, 125, zc, 0)

    def zdeg(k, carry):
      ch_id = sid + k * NS

      @pl.when(ch_id < ndz)
      def _():
        pltpu.sync_copy(zbuf, deg_sp.at[pl.ds(ch_id * 2000, 2000)])

      return carry

    lax.fori_loop(0, -(-ndz // NS), zdeg, 0)
    plsc.subcore_barrier()

    def aproc(p, sem_ab, k, blk):
      wait_stage(p)

      @pl.when(blk + NS < nbt)
      def _():
        fire_stage((blk + NS) * SBLK, 1 - p)

      @pl.when(k >= 2)
      def _():
        drain_ascatters(p, sem_ab)

      def rowloop(rr, c2):
        kr = p * CPB + rr
        for g in range(CH // 16):
          fb = rr * CH + g * 16
          sl = pl.ds(fb, 16)
          gs = pl.ds(g * 16, 16)
          key_a[kr, gs] = ets[p, sl] * n + eits[p, 1, sl]
          w = (ef4s[p, 0, sl] + ef4s[p, 1, sl] + ef4s[p, 2, sl]
               + ef4s[p, 3, sl])
          wabs_a[kr, gs] = jnp.abs(w * 0.25)
        return c2

      lax.fori_loop(0, CPB, rowloop, 0)
      for jj in range(CPB):
        pltpu.async_copy(wabs_a.at[p * CPB + jj],
                         deg_sp.at[key_a.at[p * CPB + jj]],
                         sem_ab, add=True)

    fire_stage(sid * SBLK, 0)

    def ablock(k, carry):
      blk = sid + k * NS

      @pl.when(blk < nbt)
      def _():
        @pl.when(lax.rem(k, 2) == 0)
        def _():
          aproc(0, sem_a0, k, blk)

        @pl.when(lax.rem(k, 2) == 1)
        def _():
          aproc(1, sem_a1, k, blk)

      return carry

    lax.fori_loop(0, nba, ablock, 0)
    for dk in (nba - 2, nba - 1):
      @pl.when(sid + dk * NS < nbt)
      def _(dk=dk):
        drain_ascatters(dk % 2, sem_a0 if dk % 2 == 0 else sem_a1)

    fire_stage(wid * SBLK, 0)
    plsc.subcore_barrier()

    nw = jnp.where(wid < remw, nwmax, nwmax - CPB) if remw else nwmax

    def cpipe(j, carry):
      cc = lax.rem(j, CPB)
      par = lax.rem(j, 2)

      @pl.when(jnp.logical_and(j >= 2, j - 2 < nw))
      def _():
        @pl.when(par == 0)
        def _():
          pltpu.make_async_copy(rows0, out_sp.at[srow0], sem_s0).wait()

        @pl.when(par == 1)
        def _():
          pltpu.make_async_copy(rows1, out_sp.at[srow1], sem_s1).wait()

      @pl.when(j < nw)
      def _():
        kb = lax.div(j, CPB)
        pb = lax.rem(kb, 2)

        def stage_next(p):
          wait_stage(p)

          @pl.when((kb + 1) * CPB < nw)
          def _():
            fire_stage((wid + (kb + 1) * NW) * SBLK, 1 - p)

        @pl.when(cc == 0)
        def _():
          @pl.when(pb == 0)
          def _():
            stage_next(0)

          @pl.when(pb == 1)
          def _():
            stage_next(1)

        def front(rowsb, cb, srowb, degb, sem_gb, sem_db, p):
          for g in range(CH // 16):
            fb = cc * CH + g * 16
            sl = pl.ds(fb, 16)
            gs = pl.ds(g * 16, 16)
            key_a[cc, gs] = ets[p, sl] * n + eits[p, 1, sl]
            srowb[gs] = eits[p, 0, sl]
            w = (ef4s[p, 0, sl] + ef4s[p, 1, sl] + ef4s[p, 2, sl]
                 + ef4s[p, 3, sl])
            cb[gs] = w * 0.25
          pltpu.async_copy(deg_sp.at[key_a.at[cc]], degb, sem_db)
          pltpu.async_copy(trel.at[key_a.at[cc]], rowsb, sem_gb)

        @pl.when(par == 0)
        def _():
          @pl.when(pb == 0)
          def _():
            front(rows0, c0, srow0, degc0, sem_g0, sem_d0, 0)

          @pl.when(pb == 1)
          def _():
            front(rows0, c0, srow0, degc0, sem_g0, sem_d0, 1)

        @pl.when(par == 1)
        def _():
          @pl.when(pb == 0)
          def _():
            front(rows1, c1, srow1, degc1, sem_g1, sem_d1, 0)

          @pl.when(pb == 1)
          def _():
            front(rows1, c1, srow1, degc1, sem_g1, sem_d1, 1)

      @pl.when(jnp.logical_and(j >= 1, j - 1 < nw))
      def _():
        rp = lax.rem(j - 1, CPB)

        def back(rowsb, cb, srowb, degb, sem_gb, sem_sb, sem_db):
          pltpu.make_async_copy(deg_sp.at[key_a.at[rp]], degb, sem_db).wait()
          pltpu.make_async_copy(trel.at[key_a.at[rp]], rowsb, sem_gb).wait()
          for g in range(CH // 16):
            gs = pl.ds(g * 16, 16)
            cb[gs] = cb[gs] / (degb[gs] + 1e-8)

          def scale(g, c3):
            c16 = cb[pl.ds(g * 16, 16)]
            for k in range(16):
              spl = jnp.take(c16, jnp.full((16,), k, jnp.int32), mode="fill")
              erow = g * 16 + k
              for h in range(d // 16):
                sl = pl.ds(h * 16, 16)
                rowsb[erow, sl] = rowsb[erow, sl] * spl
            return c3

          lax.fori_loop(0, CH // 16, scale, 0)
          pltpu.async_copy(rowsb, out_sp.at[srowb], sem_sb, add=True)

        @pl.when(par == 1)
        def _():
          back(rows0, c0, srow0, degc0, sem_g0, sem_s0, sem_d0)

        @pl.when(par == 0)
        def _():
          back(rows1, c1, srow1, degc1, sem_g1, sem_s1, sem_d1)

      return carry

    lax.fori_loop(0, nwmax + 2, cpipe, 0)
    plsc.subcore_barrier()

    def wb(k, carry):
      ch_id = sid + k * NS

      @pl.when(ch_id < nrc)
      def _():
        pltpu.sync_copy(out_sp.at[pl.ds(ch_id * WBC, WBC)],
                        rows0.at[pl.ds(0, WBC)])
        pltpu.sync_copy(rows0.at[pl.ds(0, WBC)],
                        out_hbm.at[cid, pl.ds(ch_id * WBC, WBC)])

      return carry

    lax.fori_loop(0, -(-nrc // NS), wb, 0)

  return sc_kernel


def kernel(node_features, edge_features, adj_e, adj_v, T, edge_index,
           edge_type, W_rel, W_self, b_self):
  n = adj_v.shape[0]
  e = edge_index.shape[1]
  din = node_features.shape[1]
  dout = W_self.shape[0]
  r = W_rel.shape[0]

  bm = 2000
  nb = n // bm
  trel = pl.pallas_call(
      _rel_transform_body,
      grid=(r, nb),
      in_specs=[
          pl.BlockSpec((bm, din), lambda q, i: (i, 0)),
          pl.BlockSpec((1, dout, din), lambda q, i: (q, 0, 0)),
      ],
      out_specs=pl.BlockSpec((bm, dout), lambda q, i: (q * nb + i, 0)),
      out_shape=jax.ShapeDtypeStruct((r * n, dout), jnp.float32),
  )(node_features, W_rel)

  self_out = pl.pallas_call(
      _self_transform_body,
      grid=(nb,),
      in_specs=[
          pl.BlockSpec((bm, din), lambda i: (i, 0)),
          pl.BlockSpec((dout, din), lambda i: (0, 0)),
          pl.BlockSpec((1, dout), lambda i: (0, 0)),
      ],
      out_specs=pl.BlockSpec((bm, dout), lambda i: (i, 0)),
      out_shape=jax.ShapeDtypeStruct((n, dout), jnp.float32),
  )(node_features, W_self, b_self.reshape(1, dout))

  ef_t = edge_features.T
  partials = _make_sc_kernel(n, e, dout, r)(
      trel, edge_type, edge_index, ef_t)

  out = pl.pallas_call(
      _combine_body,
      grid=(n // bm,),
      in_specs=[
          pl.BlockSpec((NC, bm, dout), lambda i: (0, i, 0)),
          pl.BlockSpec((bm, dout), lambda i: (i, 0)),
      ],
      out_specs=pl.BlockSpec((bm, dout), lambda i: (i, 0)),
      out_shape=jax.ShapeDtypeStruct((n, dout), jnp.float32),
  )(partials, self_out)

  return out, edge_features


# speedup vs baseline: 2.5727x; 1.0215x over previous
import functools

import jax
import jax.numpy as jnp
from jax import lax
from jax.experimental import pallas as pl
from jax.experimental.pallas import tpu as pltpu
from jax.experimental.pallas import tpu_sc as plsc

NC = 2
NS = 16
NW = NC * NS

CH = 128
SBLK = 512
CPB = SBLK // CH
WBC = 80


def _rel_transform_body(x_ref, w_ref, o_ref):
  o_ref[...] = lax.dot_general(
      x_ref[...], w_ref[0],
      dimension_numbers=(((1,), (1,)), ((), ())),
      preferred_element_type=jnp.float32,
  )


def _self_transform_body(x_ref, w_ref, b_ref, o_ref):
  o_ref[...] = lax.dot_general(
      x_ref[...], w_ref[...],
      dimension_numbers=(((1,), (1,)), ((), ())),
      preferred_element_type=jnp.float32,
  ) + b_ref[...]


def _combine_body(p_ref, s_ref, o_ref):
  o_ref[...] = p_ref[0] + p_ref[1] + s_ref[...]


def _make_sc_kernel(n, e, d, r):
  nbt = e // SBLK
  nba = -(-nbt // NS)
  nbw = -(-nbt // NW)
  remw = nbt % NW
  nwmax = nbw * CPB
  deg_sz = r * n
  nrc = n // WBC
  ndz = deg_sz // 2000

  mesh = plsc.VectorSubcoreMesh(core_axis_name="c", subcore_axis_name="s")

  @functools.partial(
      pl.kernel,
      mesh=mesh,
      compiler_params=pltpu.CompilerParams(needs_layout_passes=False),
      out_type=jax.ShapeDtypeStruct((NC, n, d), jnp.float32),
      scratch_types=[
          pltpu.VMEM_SHARED((deg_sz,), jnp.float32),
          pltpu.VMEM_SHARED((n, d), jnp.float32),
          pltpu.VMEM((2, 4, SBLK), jnp.float32),
          pltpu.VMEM((2, 2, SBLK), jnp.int32),
          pltpu.VMEM((2, SBLK), jnp.int32),
          pltpu.VMEM((2 * CPB, CH), jnp.int32),
          pltpu.VMEM((2 * CPB, CH), jnp.float32),
          pltpu.VMEM((CH, d), jnp.float32),
          pltpu.VMEM((CH, d), jnp.float32),
          pltpu.VMEM((CH,), jnp.float32),
          pltpu.VMEM((CH,), jnp.float32),
          pltpu.VMEM((CH,), jnp.int32),
          pltpu.VMEM((CH,), jnp.int32),
          pltpu.VMEM((CH,), jnp.float32),
          pltpu.VMEM((CH,), jnp.float32),
          pltpu.VMEM((2000,), jnp.float32),
          pltpu.SemaphoreType.DMA,
          pltpu.SemaphoreType.DMA,
          pltpu.SemaphoreType.DMA,
          pltpu.SemaphoreType.DMA,
          pltpu.SemaphoreType.DMA,
          pltpu.SemaphoreType.DMA,
          pltpu.SemaphoreType.DMA,
          pltpu.SemaphoreType.DMA,
          pltpu.SemaphoreType.DMA,
      ],
  )
  def sc_kernel(trel, et_h, ei_h, ef_t, out_hbm,
                deg_sp, out_sp,
                ef4s, eits, ets, key_a, wabs_a,
                rows0, rows1, c0, c1, srow0, srow1, degc0, degc1, zbuf,
                sem_g0, sem_g1, sem_s0, sem_s1, sem_a0, sem_a1, sem_b,
                sem_d0, sem_d1):
    cid = lax.axis_index("c")
    sid = lax.axis_index("s")
    wid = sid * NC + cid
    z16 = jnp.zeros((16,), jnp.float32)

    def fire_stage(off, p):
      pltpu.async_copy(ei_h.at[:, pl.ds(off, SBLK)], eits.at[p], sem_b)
      pltpu.async_copy(ef_t.at[:, pl.ds(off, SBLK)], ef4s.at[p], sem_b)
      pltpu.async_copy(et_h.at[pl.ds(off, SBLK)], ets.at[p], sem_b)

    def wait_stage(p):
      pltpu.make_async_copy(ei_h.at[:, pl.ds(0, SBLK)], eits.at[p],
                            sem_b).wait()
      pltpu.make_async_copy(ef_t.at[:, pl.ds(0, SBLK)], ef4s.at[p],
                            sem_b).wait()
      pltpu.make_async_copy(et_h.at[pl.ds(0, SBLK)], ets.at[p],
                            sem_b).wait()

    def drain_ascatters(p, sem_ab):
      for jj in range(CPB):
        pltpu.make_async_copy(wabs_a.at[p * CPB + jj],
                              deg_sp.at[key_a.at[p * CPB + jj]],
                              sem_ab).wait()

    def zrow(i, carry):
      for h in range(d // 16):
        rows0[i, pl.ds(h * 16, 16)] = z16
      return carry

    lax.fori_loop(0, CH, zrow, 0)

    def zout(k, carry):
      ch_id = sid + k * NS

      @pl.when(ch_id < nrc)
      def _():
        pltpu.sync_copy(rows0.at[pl.ds(0, WBC)],
                        out_sp.at[pl.ds(ch_id * WBC, WBC)])

      return carry

    lax.fori_loop(0, -(-nrc // NS), zout, 0)

    def zc(i, carry):
      zbuf[pl.ds(i * 16, 16)] = z16
      return carry

    lax.fori_loop(0, 125, zc, 0)

    def zdeg(k, carry):
      ch_id = sid + k * NS

      @pl.when(ch_id < ndz)
      def _():
        pltpu.sync_copy(zbuf, deg_sp.at[pl.ds(ch_id * 2000, 2000)])

      return carry

    lax.fori_loop(0, -(-ndz // NS), zdeg, 0)
    plsc.subcore_barrier()

    def aproc(p, sem_ab, k, blk):
      wait_stage(p)

      @pl.when(blk + NS < nbt)
      def _():
        fire_stage((blk + NS) * SBLK, 1 - p)

      @pl.when(k >= 2)
      def _():
        drain_ascatters(p, sem_ab)

      def rowloop(rr, c2):
        kr = p * CPB + rr
        for g in range(CH // 16):
          fb = rr * CH + g * 16
          sl = pl.ds(fb, 16)
          gs = pl.ds(g * 16, 16)
          key_a[kr, gs] = ets[p, sl] * n + eits[p, 1, sl]
          w = (ef4s[p, 0, sl] + ef4s[p, 1, sl] + ef4s[p, 2, sl]
               + ef4s[p, 3, sl])
          wabs_a[kr, gs] = jnp.abs(w * 0.25)
        return c2

      lax.fori_loop(0, CPB, rowloop, 0)
      for jj in range(CPB):
        pltpu.async_copy(wabs_a.at[p * CPB + jj],
                         deg_sp.at[key_a.at[p * CPB + jj]],
                         sem_ab, add=True)

    fire_stage(sid * SBLK, 0)

    def ablock(k, carry):
      blk = sid + k * NS

      @pl.when(blk < nbt)
      def _():
        @pl.when(lax.rem(k, 2) == 0)
        def _():
          aproc(0, sem_a0, k, blk)

        @pl.when(lax.rem(k, 2) == 1)
        def _():
          aproc(1, sem_a1, k, blk)

      return carry

    lax.fori_loop(0, nba, ablock, 0)
    for dk in (nba - 2, nba - 1):
      @pl.when(sid + dk * NS < nbt)
      def _(dk=dk):
        drain_ascatters(dk % 2, sem_a0 if dk % 2 == 0 else sem_a1)

    fire_stage(wid * SBLK, 0)
    plsc.subcore_barrier()

    nw = jnp.where(wid < remw, nwmax, nwmax - CPB) if remw else nwmax

    def cpipe(j, carry):
      cc = lax.rem(j, CPB)
      par = lax.rem(j, 2)

      @pl.when(jnp.logical_and(j >= 2, j - 2 < nw))
      def _():
        @pl.when(par == 0)
        def _():
          pltpu.make_async_copy(rows0, out_sp.at[srow0], sem_s0).wait()

        @pl.when(par == 1)
        def _():
          pltpu.make_async_copy(rows1, out_sp.at[srow1], sem_s1).wait()

      @pl.when(j < nw)
      def _():
        kb = lax.div(j, CPB)
        pb = lax.rem(kb, 2)

        def stage_next(p):
          wait_stage(p)

          @pl.when((kb + 1) * CPB < nw)
          def _():
            fire_stage((wid + (kb + 1) * NW) * SBLK, 1 - p)

        @pl.when(cc == 0)
        def _():
          @pl.when(pb == 0)
          def _():
            stage_next(0)

          @pl.when(pb == 1)
          def _():
            stage_next(1)

        def front(rowsb, cb, srowb, degb, sem_gb, sem_db, p):
          for g in range(CH // 16):
            fb = cc * CH + g * 16
            sl = pl.ds(fb, 16)
            gs = pl.ds(g * 16, 16)
            key_a[cc, gs] = ets[p, sl] * n + eits[p, 1, sl]
            srowb[gs] = eits[p, 0, sl]
            w = (ef4s[p, 0, sl] + ef4s[p, 1, sl] + ef4s[p, 2, sl]
                 + ef4s[p, 3, sl])
            cb[gs] = w * 0.25
          pltpu.async_copy(deg_sp.at[key_a.at[cc]], degb, sem_db)
          pltpu.async_copy(trel.at[key_a.at[cc]], rowsb, sem_gb)

        @pl.when(par == 0)
        def _():
          @pl.when(pb == 0)
          def _():
            front(rows0, c0, srow0, degc0, sem_g0, sem_d0, 0)

          @pl.when(pb == 1)
          def _():
            front(rows0, c0, srow0, degc0, sem_g0, sem_d0, 1)

        @pl.when(par == 1)
        def _():
          @pl.when(pb == 0)
          def _():
            front(rows1, c1, srow1, degc1, sem_g1, sem_d1, 0)

          @pl.when(pb == 1)
          def _():
            front(rows1, c1, srow1, degc1, sem_g1, sem_d1, 1)

      @pl.when(jnp.logical_and(j >= 1, j - 1 < nw))
      def _():
        rp = lax.rem(j - 1, CPB)

        def back(rowsb, cb, srowb, degb, sem_gb, sem_sb, sem_db):
          pltpu.make_async_copy(deg_sp.at[key_a.at[rp]], degb, sem_db).wait()
          pltpu.make_async_copy(trel.at[key_a.at[rp]], rowsb, sem_gb).wait()
          for g in range(CH // 16):
            gs = pl.ds(g * 16, 16)
            cb[gs] = cb[gs] / (degb[gs] + 1e-8)

          def scale(g, c3):
            c16 = cb[pl.ds(g * 16, 16)]
            for k in range(16):
              spl = jnp.take(c16, jnp.full((16,), k, jnp.int32), mode="fill")
              erow = g * 16 + k
              for h in range(d // 16):
                sl = pl.ds(h * 16, 16)
                rowsb[erow, sl] = rowsb[erow, sl] * spl
            return c3

          lax.fori_loop(0, CH // 16, scale, 0)
          pltpu.async_copy(rowsb, out_sp.at[srowb], sem_sb, add=True)

        @pl.when(par == 1)
        def _():
          back(rows0, c0, srow0, degc0, sem_g0, sem_s0, sem_d0)

        @pl.when(par == 0)
        def _():
          back(rows1, c1, srow1, degc1, sem_g1, sem_s1, sem_d1)

      return carry

    lax.fori_loop(0, nwmax + 2, cpipe, 0)
    plsc.subcore_barrier()

    def wb(k, carry):
      ch_id = sid + k * NS

      @pl.when(ch_id < nrc)
      def _():
        pltpu.sync_copy(out_sp.at[pl.ds(ch_id * WBC, WBC)],
                        rows0.at[pl.ds(0, WBC)])
        pltpu.sync_copy(rows0.at[pl.ds(0, WBC)],
                        out_hbm.at[cid, pl.ds(ch_id * WBC, WBC)])

      return carry

    lax.fori_loop(0, -(-nrc // NS), wb, 0)

  return sc_kernel


def kernel(node_features, edge_features, adj_e, adj_v, T, edge_index,
           edge_type, W_rel, W_self, b_self):
  n = adj_v.shape[0]
  e = edge_index.shape[1]
  din = node_features.shape[1]
  dout = W_self.shape[0]
  r = W_rel.shape[0]

  bm = 2000
  nb = n // bm
  trel = pl.pallas_call(
      _rel_transform_body,
      grid=(nb, r),
      in_specs=[
          pl.BlockSpec((bm, din), lambda i, q: (i, 0)),
          pl.BlockSpec((1, dout, din), lambda i, q: (q, 0, 0)),
      ],
      out_specs=pl.BlockSpec((bm, dout), lambda i, q: (q * nb + i, 0)),
      out_shape=jax.ShapeDtypeStruct((r * n, dout), jnp.float32),
  )(node_features, W_rel)

  self_out = pl.pallas_call(
      _self_transform_body,
      grid=(nb,),
      in_specs=[
          pl.BlockSpec((bm, din), lambda i: (i, 0)),
          pl.BlockSpec((dout, din), lambda i: (0, 0)),
          pl.BlockSpec((1, dout), lambda i: (0, 0)),
      ],
      out_specs=pl.BlockSpec((bm, dout), lambda i: (i, 0)),
      out_shape=jax.ShapeDtypeStruct((n, dout), jnp.float32),
  )(node_features, W_self, b_self.reshape(1, dout))

  ef_t = edge_features.T
  partials = _make_sc_kernel(n, e, dout, r)(
      trel, edge_type, edge_index, ef_t)

  out = pl.pallas_call(
      _combine_body,
      grid=(n // bm,),
      in_specs=[
          pl.BlockSpec((NC, bm, dout), lambda i: (0, i, 0)),
          pl.BlockSpec((bm, dout), lambda i: (i, 0)),
      ],
      out_specs=pl.BlockSpec((bm, dout), lambda i: (i, 0)),
      out_shape=jax.ShapeDtypeStruct((n, dout), jnp.float32),
  )(partials, self_out)

  return out, edge_features
